# baseline (speedup 1.0000x reference)
import jax
import jax.numpy as jnp
from jax.experimental import pallas as pl
from jax.experimental.pallas import tpu as pltpu

_TAU = 0.3
_BB = 16
_LOG2E = 1.4426950408889634


def _body(logits_ref, u_ref, out_ref):
    lgt2 = logits_ref[...] * (_LOG2E / _TAU)
    u = u_ref[...]
    w = -jnp.log(u)
    x2 = lgt2[:, None, :] - jnp.log(w) * (_LOG2E / _TAU)
    m2 = jnp.max(x2, axis=2, keepdims=True)
    e = jnp.exp2(x2 - m2)
    s = jnp.sum(e, axis=2, keepdims=True)
    p = e / s
    out_ref[...] = jnp.max(p.transpose(1, 0, 2), axis=0)


def kernel(logits, uniform):
    B, D = logits.shape
    K = uniform.shape[1]
    return pl.pallas_call(
        _body,
        grid=(B // _BB,),
        in_specs=[
            pl.BlockSpec((_BB, D), lambda b: (b, 0)),
            pl.BlockSpec((_BB, K, D), lambda b: (b, 0, 0)),
        ],
        out_specs=pl.BlockSpec((_BB, D), lambda b: (b, 0)),
        out_shape=jax.ShapeDtypeStruct((B, D), jnp.float32),
        compiler_params=pltpu.CompilerParams(
            dimension_semantics=("parallel",)),
    )(logits, uniform)


# speedup vs baseline: 1.0515x; 1.0515x over previous
import jax
import jax.numpy as jnp
from jax.experimental import pallas as pl
from jax.experimental.pallas import tpu as pltpu

_TAU = 0.3
_BB = 8
_LOG2E = 1.4426950408889634


def _body(logits_ref, u_ref, out_ref):
    lgt2 = logits_ref[...] * (_LOG2E / _TAU)
    ut = u_ref[...].transpose(1, 0, 2)
    acc = None
    for k in range(ut.shape[0]):
        u = ut[k]
        w = -jnp.log(u)
        x2 = lgt2 - jnp.log(w) * (_LOG2E / _TAU)
        m2 = jnp.max(x2, axis=1, keepdims=True)
        e = jnp.exp2(x2 - m2)
        s = jnp.sum(e, axis=1, keepdims=True)
        p = e / s
        acc = p if acc is None else jnp.maximum(acc, p)
    out_ref[...] = acc


def kernel(logits, uniform):
    B, D = logits.shape
    K = uniform.shape[1]
    return pl.pallas_call(
        _body,
        grid=(B // _BB,),
        in_specs=[
            pl.BlockSpec((_BB, D), lambda b: (b, 0)),
            pl.BlockSpec((_BB, K, D), lambda b: (b, 0, 0)),
        ],
        out_specs=pl.BlockSpec((_BB, D), lambda b: (b, 0)),
        out_shape=jax.ShapeDtypeStruct((B, D), jnp.float32),
        compiler_params=pltpu.CompilerParams(
            dimension_semantics=("parallel",)),
    )(logits, uniform)
